# Initial kernel scaffold; baseline (speedup 1.0000x reference)
import jax
import jax.numpy as jnp
from jax.experimental import pallas as pl


def kernel(sub, rel, obj, edge_index, edge_type, init_embed, init_rel, rel_wt1, w_in1, w_out1, w_loop1, w_rel1, loop_rel1, gamma1, beta1, w_in2, w_out2, w_loop2, w_rel2, loop_rel2, gamma2, beta2):
    raise NotImplementedError("write your pallas kernel here")


# speedup vs baseline: 6.0168x; 6.0168x over previous
import functools

import jax
import jax.numpy as jnp
from jax import lax
from jax.experimental import pallas as pl
from jax.experimental.pallas import tpu as pltpu
from jax.experimental.pallas import tpu_sc as plsc

N = 10000
D = 128
NE = 320000
NED = NE // 2
NRALL = 201
BATCH = 4096
NS = 16
NC = 2
B = 128
NB = 79
EPT = NB * B
E_PAD = NS * EPT
RPT = 640
N_P = NS * RPT
XS_ROWS = NC * N_P
BR = 2048
NSTEP = N_P // BR

_f32 = jnp.float32
_i32 = jnp.int32
_HP = jax.lax.Precision.HIGHEST
_mesh = plsc.VectorSubcoreMesh(core_axis_name="c", subcore_axis_name="s")


def _sds(shape, dtype=_f32):
    return jax.ShapeDtypeStruct(shape, dtype)


@functools.partial(
    pl.kernel,
    out_type=_sds((NC, N_P, D)),
    mesh=_mesh,
    scratch_types=[
        pltpu.VMEM((B,), _i32),
        pltpu.VMEM((B, D), _f32),
        pltpu.VMEM((B, D), _f32),
        pltpu.VMEM_SHARED((N_P, D), _f32),
    ],
)
def _deg_kernel(didx_hbm, deg_hbm, idx_v, ones_v, zero_v, dacc):
    cid = lax.axis_index("c")
    sid = lax.axis_index("s")

    @pl.loop(0, B)
    def _(i):
        for c in range(D // 16):
            sl = pl.ds(c * 16, 16)
            ones_v[i, sl] = jnp.full((16,), 1.0, _f32)
            zero_v[i, sl] = jnp.zeros((16,), _f32)

    for k in range(RPT // B):
        pltpu.sync_copy(zero_v, dacc.at[pl.ds(sid * RPT + k * B, B)])
    plsc.subcore_barrier()

    @pl.loop(0, NB)
    def _(b):
        base = sid * EPT + b * B
        pltpu.sync_copy(didx_hbm.at[cid, pl.ds(base, B)], idx_v)
        pltpu.sync_copy(ones_v, dacc.at[idx_v], add=True)

    plsc.subcore_barrier()
    for k in range(RPT // B):
        r0 = sid * RPT + k * B
        pltpu.sync_copy(dacc.at[pl.ds(r0, B)], deg_hbm.at[cid, pl.ds(r0, B)])


@functools.partial(
    pl.kernel,
    out_type=_sds((NC, N_P, D)),
    mesh=_mesh,
    scratch_types=[
        pltpu.VMEM((B,), _i32),
        pltpu.VMEM((B,), _i32),
        pltpu.VMEM((B,), _i32),
        pltpu.VMEM((B, D), _f32),
        pltpu.VMEM((B, D), _f32),
        pltpu.VMEM_SHARED((N_P, D), _f32),
        pltpu.SemaphoreType.DMA,
        pltpu.SemaphoreType.DMA,
    ],
)
def _msg_kernel(xs_hbm, rel_hbm, src_hbm, et_hbm, dst_hbm, acc_hbm,
                src_v, et_v, dst_v, xrows, rrows, acc, sem1, sem2):
    cid = lax.axis_index("c")
    sid = lax.axis_index("s")

    @pl.loop(0, B)
    def _(i):
        for c in range(D // 16):
            xrows[i, pl.ds(c * 16, 16)] = jnp.zeros((16,), _f32)

    for k in range(RPT // B):
        pltpu.sync_copy(xrows, acc.at[pl.ds(sid * RPT + k * B, B)])
    plsc.subcore_barrier()

    @pl.loop(0, NB)
    def _(b):
        base = sid * EPT + b * B
        pltpu.sync_copy(src_hbm.at[cid, pl.ds(base, B)], src_v)
        pltpu.sync_copy(et_hbm.at[cid, pl.ds(base, B)], et_v)
        pltpu.sync_copy(dst_hbm.at[cid, pl.ds(base, B)], dst_v)
        g1 = pltpu.async_copy(xs_hbm.at[src_v], xrows, sem1)
        g2 = pltpu.async_copy(rel_hbm.at[et_v], rrows, sem2)
        g1.wait()
        g2.wait()

        @pl.loop(0, B)
        def _(e):
            for c in range(D // 16):
                sl = pl.ds(c * 16, 16)
                xrows[e, sl] = xrows[e, sl] * rrows[e, sl]

        pltpu.sync_copy(xrows, acc.at[dst_v], add=True)

    plsc.subcore_barrier()
    for k in range(RPT // B):
        r0 = sid * RPT + k * B
        pltpu.sync_copy(acc.at[pl.ds(r0, B)], acc_hbm.at[cid, pl.ds(r0, B)])


@functools.partial(
    pl.kernel,
    out_type=[_sds((BATCH, D)), _sds((BATCH, D)), _sds((BATCH, D))],
    mesh=_mesh,
    scratch_types=[
        pltpu.VMEM((B,), _i32),
        pltpu.VMEM((B, D), _f32),
        pltpu.SemaphoreType.DMA,
    ],
)
def _gather_kernel(x_hbm, r_hbm, sub_hbm, rel_hbm, obj_hbm,
                   sub_o, rel_o, obj_o, idx_v, rows_v, sem):
    cid = lax.axis_index("c")
    sid = lax.axis_index("s")
    base = (sid * NC + cid) * B
    for tab, ih, oh in ((x_hbm, sub_hbm, sub_o),
                        (r_hbm, rel_hbm, rel_o),
                        (x_hbm, obj_hbm, obj_o)):
        pltpu.sync_copy(ih.at[pl.ds(base, B)], idx_v)
        pltpu.async_copy(tab.at[idx_v], rows_v, sem).wait()
        pltpu.sync_copy(rows_v, oh.at[pl.ds(base, B)])


def _deg_inv(deg):
    return jnp.where(deg > 0.0, 1.0 / jnp.sqrt(jnp.maximum(deg, 1e-12)), 0.0)


def _dot(a, b):
    return jnp.dot(a, b, preferred_element_type=_f32, precision=_HP)


def _prep_body(x_ref, deg_ref, relwt_ref, initrel_ref, looprel_ref,
               xs_ref, relall_ref):
    i = pl.program_id(0)
    di = _deg_inv(deg_ref[0, :, 0])
    do = _deg_inv(deg_ref[1, :, 0])
    x = x_ref[...]
    xs_ref[0, :, :] = x * di[:, None]
    xs_ref[1, :, :] = x * do[:, None]

    @pl.when(i == 0)
    def _():
        rel = _dot(relwt_ref[...], initrel_ref[...])
        relall_ref[...] = jnp.concatenate([rel, looprel_ref[...]], axis=0)


_prep = pl.pallas_call(
    _prep_body,
    grid=(NSTEP,),
    in_specs=[
        pl.BlockSpec((BR, D), lambda i: (i, 0)),
        pl.BlockSpec((2, BR, D), lambda i: (0, i, 0)),
        pl.BlockSpec((NRALL - 1, 5), lambda i: (0, 0)),
        pl.BlockSpec((5, D), lambda i: (0, 0)),
        pl.BlockSpec((1, D), lambda i: (0, 0)),
    ],
    out_specs=[
        pl.BlockSpec((2, BR, D), lambda i: (0, i, 0)),
        pl.BlockSpec((NRALL, D), lambda i: (0, 0)),
    ],
    out_shape=[_sds((NC, N_P, D)), _sds((NRALL, D))],
)


def _conv_body(final, acc_ref, x_ref, deg_ref, w_in_ref, w_out_ref,
               w_loop_ref, relall_ref, w_rel_ref, gamma_ref, beta_ref,
               looprel_next_ref, *rest):
    if final:
        x_out_ref, r_out_ref, stats = rest
    else:
        x_out_ref, xs_ref, relall2_ref, stats = rest
    p = pl.program_id(0)
    i = pl.program_id(1)
    di = _deg_inv(deg_ref[0, :, 0])
    do = _deg_inv(deg_ref[1, :, 0])
    x = x_ref[...]
    loop_row = relall_ref[NRALL - 1:NRALL, :]
    out = (_dot(acc_ref[0, :, :] * di[:, None], w_in_ref[...])
           + _dot(acc_ref[1, :, :] * do[:, None], w_out_ref[...])
           + _dot(x * loop_row, w_loop_ref[...])) / 3.0

    @pl.when(p == 0)
    def _():
        @pl.when(i == 0)
        def _():
            stats[...] = jnp.zeros((8, D), _f32)

        stats[0, :] += jnp.sum(out, axis=0)
        stats[1, :] += jnp.sum(out * out, axis=0)

    @pl.when(p == 1)
    def _():
        mean = stats[0, :] / float(N)
        var = stats[1, :] / float(N) - mean * mean
        xn = (out - mean[None, :]) / jnp.sqrt(var[None, :] + 1e-5)
        xnew = jnp.tanh(xn * gamma_ref[...][None, :] + beta_ref[...][None, :])
        rows = i * BR + lax.broadcasted_iota(_i32, (BR, 1), 0)
        xnew = jnp.where(rows < N, xnew, 0.0)
        x_out_ref[...] = xnew
        if not final:
            xs_ref[0, :, :] = xnew * di[:, None]
            xs_ref[1, :, :] = xnew * do[:, None]

        @pl.when(i == 0)
        def _():
            rel_next = _dot(relall_ref[...], w_rel_ref[...])
            if final:
                r_out_ref[...] = rel_next[:NRALL - 1, :]
            else:
                relall2_ref[...] = jnp.concatenate(
                    [rel_next[:NRALL - 1, :], looprel_next_ref[...]], axis=0)


def _make_conv(final):
    in_specs = [
        pl.BlockSpec((2, BR, D), lambda p, i: (0, i, 0)),
        pl.BlockSpec((BR, D), lambda p, i: (i, 0)),
        pl.BlockSpec((2, BR, D), lambda p, i: (0, i, 0)),
        pl.BlockSpec((D, D), lambda p, i: (0, 0)),
        pl.BlockSpec((D, D), lambda p, i: (0, 0)),
        pl.BlockSpec((D, D), lambda p, i: (0, 0)),
        pl.BlockSpec((NRALL, D), lambda p, i: (0, 0)),
        pl.BlockSpec((D, D), lambda p, i: (0, 0)),
        pl.BlockSpec((D,), lambda p, i: (0,)),
        pl.BlockSpec((D,), lambda p, i: (0,)),
        pl.BlockSpec((1, D), lambda p, i: (0, 0)),
    ]
    if final:
        out_specs = [
            pl.BlockSpec((BR, D), lambda p, i: (i, 0)),
            pl.BlockSpec((NRALL - 1, D), lambda p, i: (0, 0)),
        ]
        out_shape = [_sds((N_P, D)), _sds((NRALL - 1, D))]
    else:
        out_specs = [
            pl.BlockSpec((BR, D), lambda p, i: (i, 0)),
            pl.BlockSpec((2, BR, D), lambda p, i: (0, i, 0)),
            pl.BlockSpec((NRALL, D), lambda p, i: (0, 0)),
        ]
        out_shape = [_sds((N_P, D)), _sds((NC, N_P, D)), _sds((NRALL, D))]
    return pl.pallas_call(
        functools.partial(_conv_body, final),
        grid=(2, NSTEP),
        in_specs=in_specs,
        out_specs=out_specs,
        out_shape=out_shape,
        scratch_shapes=[pltpu.VMEM((8, D), _f32)],
    )


_conv_mid = _make_conv(False)
_conv_fin = _make_conv(True)


def kernel(sub, rel, obj, edge_index, edge_type, init_embed, init_rel,
           rel_wt1, w_in1, w_out1, w_loop1, w_rel1, loop_rel1, gamma1, beta1,
           w_in2, w_out2, w_loop2, w_rel2, loop_rel2, gamma2, beta2):
    ei = edge_index.astype(_i32)
    et = edge_type.astype(_i32)
    npad = E_PAD - NED

    def _pad(a, v):
        return jnp.concatenate([a, jnp.full((npad,), v, _i32)])

    in_src = _pad(ei[0, :NED], N)
    in_dst = _pad(ei[1, :NED], N)
    in_t = _pad(et[:NED], 0)
    out_src = _pad(ei[0, NED:], N)
    out_dst = _pad(ei[1, NED:], N)
    out_t = _pad(et[NED:], 0)

    didx2 = jnp.stack([in_src, out_src])
    src2 = jnp.stack([in_src, out_src + N_P])
    dst2 = jnp.stack([in_dst, out_dst])
    et2 = jnp.stack([in_t, out_t])

    x0 = jnp.concatenate([init_embed, jnp.zeros((N_P - N, D), _f32)], axis=0)

    deg2 = _deg_kernel(didx2)
    xs1, relall1 = _prep(x0, deg2, rel_wt1, init_rel, loop_rel1)
    acc1 = _msg_kernel(xs1.reshape(XS_ROWS, D), relall1, src2, et2, dst2)
    x1, xs2, relall2 = _conv_mid(acc1, x0, deg2, w_in1, w_out1, w_loop1,
                                 relall1, w_rel1, gamma1, beta1, loop_rel2)
    acc2 = _msg_kernel(xs2.reshape(XS_ROWS, D), relall2, src2, et2, dst2)
    x2, r_out = _conv_fin(acc2, x1, deg2, w_in2, w_out2, w_loop2, relall2,
                          w_rel2, gamma2, beta2, loop_rel2)
    return tuple(_gather_kernel(x2, r_out, sub.astype(_i32), rel.astype(_i32),
                                obj.astype(_i32)))
